# baseline (speedup 1.0000x reference)
import functools

import jax
import jax.numpy as jnp
from jax import lax
from jax.experimental import pallas as pl
from jax.experimental.pallas import tpu as pltpu
from jax.experimental.pallas import tpu_sc as plsc

_LANES = 16


@functools.lru_cache(maxsize=None)
def _build(batch, seq_len, vocab, hidden, ids_stride, split):
    info = plsc.get_sparse_core_info()
    num_workers = info.num_cores * info.num_subcores
    assert batch % num_workers == 0
    seq_per_worker = batch // num_workers
    assert hidden % _LANES == 0
    vecs_per_row = hidden // _LANES
    rest = seq_len - split
    phase_rows = (split, split)
    phase_add = (split, rest)
    phase_gather = (split, split)
    assert rest <= split and split % 8 == 0 and ids_stride == 2 * split

    mesh = plsc.VectorSubcoreMesh(core_axis_name="c", subcore_axis_name="s")

    def body(ids_hbm, table_hbm, pos_hbm, out_hbm, idx_v, pos_v, buf, gsem, ssem):
        wid = lax.axis_index("s") * info.num_cores + lax.axis_index("c")
        seq0 = wid * seq_per_worker
        pltpu.sync_copy(
            ids_hbm.at[pl.ds(seq0 * ids_stride, seq_per_worker * ids_stride)], idx_v
        )
        pltpu.sync_copy(pos_hbm, pos_v)

        def gather(b, ph):
            idx_slice = idx_v.at[pl.ds(b * ids_stride + ph * split, phase_gather[ph])]
            return pltpu.make_async_copy(
                table_hbm.at[idx_slice],
                buf.at[ph, pl.ds(0, phase_gather[ph])],
                gsem.at[ph],
            )

        def scatter(b, ph):
            return pltpu.make_async_copy(
                buf.at[ph, pl.ds(0, phase_rows[ph])],
                out_hbm.at[seq0 + b, pl.ds(ph * split, phase_rows[ph])],
                ssem.at[ph],
            )

        def add_pos(ph):
            @plsc.parallel_loop(0, phase_add[ph], unroll=2)
            def row_loop(i):
                for j in range(vecs_per_row):
                    sl = pl.ds(j * _LANES, _LANES)
                    plsc.addupdate(buf.at[ph, i, sl], pos_v[ph * split + i, sl])

        gather(0, 0).start()

        @pl.loop(0, seq_per_worker)
        def seq_loop(b):
            @pl.when(b >= 1)
            def _():
                scatter(b - 1, 1).wait()

            gather(b, 1).start()
            gather(b, 0).wait()
            add_pos(0)
            scatter(b, 0).start()

            @pl.when(b + 1 < seq_per_worker)
            def _():
                scatter(b, 0).wait()
                gather(b + 1, 0).start()

            gather(b, 1).wait()
            add_pos(1)
            scatter(b, 1).start()

        scatter(seq_per_worker - 1, 0).wait()
        scatter(seq_per_worker - 1, 1).wait()

    return pl.kernel(
        body,
        out_type=jax.ShapeDtypeStruct((batch, ids_stride, hidden), jnp.float32),
        mesh=mesh,
        scratch_types=[
            pltpu.VMEM((seq_per_worker * ids_stride,), jnp.int32),
            pltpu.VMEM((seq_len, hidden), jnp.float32),
            pltpu.VMEM((2, split, hidden), jnp.float32),
            pltpu.SemaphoreType.DMA((2,)),
            pltpu.SemaphoreType.DMA((2,)),
        ],
    )


def kernel(input_ids, token_table, pos_table):
    batch, seq_len = input_ids.shape
    vocab, hidden = token_table.shape
    assert seq_len == pos_table.shape[0]
    split = 40
    ids_stride = 2 * split
    pad = ids_stride - seq_len
    ids = jnp.pad(input_ids.astype(jnp.int32), ((0, 0), (0, pad))).reshape(-1)
    fn = _build(batch, seq_len, vocab, hidden, ids_stride, split)
    out = fn(ids, token_table, pos_table)
    return out[:, :seq_len, :]


# speedup vs baseline: 1.0108x; 1.0108x over previous
import functools

import jax
import jax.numpy as jnp
from jax import lax
from jax.experimental import pallas as pl
from jax.experimental.pallas import tpu as pltpu
from jax.experimental.pallas import tpu_sc as plsc

_LANES = 16


@functools.lru_cache(maxsize=None)
def _build(batch, seq_len, vocab, hidden, ids_stride, split):
    info = plsc.get_sparse_core_info()
    num_workers = info.num_cores * info.num_subcores
    assert batch % num_workers == 0
    seq_per_worker = batch // num_workers
    assert hidden % _LANES == 0
    vecs_per_row = hidden // _LANES
    n_full = seq_len // split
    tail = ids_stride - n_full * split
    offs = tuple(i * split for i in range(n_full)) + (n_full * split,)
    sizes = (split,) * n_full + (tail,)
    adds = (split,) * n_full + (seq_len - n_full * split,)
    n_ph = len(offs)
    assert split % 8 == 0 and tail % 8 == 0 and 0 < tail
    nbuf = 3

    mesh = plsc.VectorSubcoreMesh(core_axis_name="c", subcore_axis_name="s")

    def body(ids_hbm, table_hbm, pos_hbm, out_hbm, idx_v, pos_v, buf, gsem, ssem):
        wid = lax.axis_index("s") * info.num_cores + lax.axis_index("c")
        seq0 = wid * seq_per_worker
        pltpu.sync_copy(
            ids_hbm.at[pl.ds(seq0 * ids_stride, seq_per_worker * ids_stride)], idx_v
        )
        pltpu.sync_copy(pos_hbm, pos_v)

        def gather(b, ph, slot):
            idx_slice = idx_v.at[pl.ds(b * ids_stride + offs[ph], sizes[ph])]
            return pltpu.make_async_copy(
                table_hbm.at[idx_slice],
                buf.at[slot, pl.ds(0, sizes[ph])],
                gsem.at[slot],
            )

        def scatter(b, ph, slot):
            return pltpu.make_async_copy(
                buf.at[slot, pl.ds(0, sizes[ph])],
                out_hbm.at[seq0 + b, pl.ds(offs[ph], sizes[ph])],
                ssem.at[slot],
            )

        def add_pos(ph, slot):
            @plsc.parallel_loop(0, adds[ph], unroll=2)
            def row_loop(i):
                for j in range(vecs_per_row):
                    sl = pl.ds(j * _LANES, _LANES)
                    plsc.addupdate(buf.at[slot, i, sl], pos_v[offs[ph] + i, sl])

        gather(0, 0, 0).start()

        @pl.loop(0, seq_per_worker)
        def seq_loop(b):
            for ph in range(n_ph):
                k = n_ph * b + ph
                slot = lax.rem(k, nbuf)
                slot_next = lax.rem(k + 1, nbuf)

                if ph >= 2:
                    scatter(b, ph - 2, slot_next).wait()
                else:

                    @pl.when(b >= 1)
                    def _(b=b, ph=ph, slot_next=slot_next):
                        scatter(b - 1, ph + n_ph - 2, slot_next).wait()

                if ph + 1 < n_ph:
                    gather(b, ph + 1, slot_next).start()
                else:

                    @pl.when(b + 1 < seq_per_worker)
                    def _(b=b, slot_next=slot_next):
                        gather(b + 1, 0, slot_next).start()

                gather(b, ph, slot).wait()
                add_pos(ph, slot)
                scatter(b, ph, slot).start()

        last = n_ph * seq_per_worker - 1
        scatter(seq_per_worker - 1, n_ph - 2, lax.rem(last - 1, nbuf)).wait()
        scatter(seq_per_worker - 1, n_ph - 1, lax.rem(last, nbuf)).wait()

    return pl.kernel(
        body,
        out_type=jax.ShapeDtypeStruct((batch, ids_stride, hidden), jnp.float32),
        mesh=mesh,
        scratch_types=[
            pltpu.VMEM((seq_per_worker * ids_stride,), jnp.int32),
            pltpu.VMEM((seq_len, hidden), jnp.float32),
            pltpu.VMEM((nbuf, split, hidden), jnp.float32),
            pltpu.SemaphoreType.DMA((nbuf,)),
            pltpu.SemaphoreType.DMA((nbuf,)),
        ],
    )


def kernel(input_ids, token_table, pos_table):
    batch, seq_len = input_ids.shape
    vocab, hidden = token_table.shape
    assert seq_len == pos_table.shape[0]
    split = 24
    ids_stride = 80
    pad = ids_stride - seq_len
    ids = jnp.pad(input_ids.astype(jnp.int32), ((0, 0), (0, pad))).reshape(-1)
    fn = _build(batch, seq_len, vocab, hidden, ids_stride, split)
    out = fn(ids, token_table, pos_table)
    return out[:, :seq_len, :]


# speedup vs baseline: 3.0888x; 3.0558x over previous
import functools

import jax
import jax.numpy as jnp
from jax import lax
from jax.experimental import pallas as pl
from jax.experimental.pallas import tpu as pltpu
from jax.experimental.pallas import tpu_sc as plsc

_LANES = 16
_JBLK = 8


@functools.lru_cache(maxsize=None)
def _build(batch, seq_len, vocab, hidden):
    info = plsc.get_sparse_core_info()
    num_workers = info.num_cores * info.num_subcores
    assert batch % num_workers == 0
    blk = batch // num_workers
    assert blk % 8 == 0
    ids_per_worker = blk * seq_len
    assert hidden % (_LANES * _JBLK) == 0
    vecs_per_row = hidden // _LANES

    mesh = plsc.VectorSubcoreMesh(core_axis_name="c", subcore_axis_name="s")

    def body(ids_hbm, table_hbm, pos_hbm, out_hbm, idx_v, pos_v, buf, gsem, ssem):
        wid = lax.axis_index("s") * info.num_cores + lax.axis_index("c")
        b0 = wid * blk
        pltpu.sync_copy(
            ids_hbm.at[pl.ds(wid * ids_per_worker, ids_per_worker)], idx_v
        )
        pltpu.sync_copy(pos_hbm, pos_v)

        def gather(l, slot):
            idx_slice = idx_v.at[pl.ds(l * blk, blk)]
            return pltpu.make_async_copy(
                table_hbm.at[idx_slice], buf.at[slot], gsem.at[slot]
            )

        def scatter(l, slot):
            return pltpu.make_async_copy(
                buf.at[slot], out_hbm.at[l, pl.ds(b0, blk)], ssem.at[slot]
            )

        gather(0, 0).start()

        @pl.loop(0, seq_len)
        def pos_loop(l):
            slot = lax.rem(l, 2)
            other = 1 - slot

            @pl.when(l >= 1)
            def _():
                scatter(l - 1, other).wait()

            @pl.when(l + 1 < seq_len)
            def _():
                gather(l + 1, other).start()

            gather(l, slot).wait()

            for jb in range(vecs_per_row // _JBLK):
                pvs = [
                    pos_v[l, pl.ds((jb * _JBLK + j) * _LANES, _LANES)]
                    for j in range(_JBLK)
                ]

                @plsc.parallel_loop(0, blk, unroll=2)
                def row_loop(i, pvs=pvs, jb=jb):
                    for j in range(_JBLK):
                        sl = pl.ds((jb * _JBLK + j) * _LANES, _LANES)
                        plsc.addupdate(buf.at[slot, i, sl], pvs[j])

            scatter(l, slot).start()

        scatter(seq_len - 1, lax.rem(seq_len - 1, 2)).wait()

    return pl.kernel(
        body,
        out_type=jax.ShapeDtypeStruct((seq_len, batch, hidden), jnp.float32),
        mesh=mesh,
        scratch_types=[
            pltpu.VMEM((ids_per_worker,), jnp.int32),
            pltpu.VMEM((seq_len, hidden), jnp.float32),
            pltpu.VMEM((2, blk, hidden), jnp.float32),
            pltpu.SemaphoreType.DMA((2,)),
            pltpu.SemaphoreType.DMA((2,)),
        ],
    )


def kernel(input_ids, token_table, pos_table):
    batch, seq_len = input_ids.shape
    vocab, hidden = token_table.shape
    assert seq_len == pos_table.shape[0]
    info = plsc.get_sparse_core_info()
    num_workers = info.num_cores * info.num_subcores
    blk = batch // num_workers
    ids = (
        input_ids.astype(jnp.int32)
        .reshape(num_workers, blk, seq_len)
        .transpose(0, 2, 1)
        .reshape(-1)
    )
    fn = _build(batch, seq_len, vocab, hidden)
    out = fn(ids, token_table, pos_table)
    return out.transpose(1, 0, 2)


# speedup vs baseline: 3.1844x; 1.0309x over previous
import functools

import jax
import jax.numpy as jnp
from jax import lax
from jax.experimental import pallas as pl
from jax.experimental.pallas import tpu as pltpu
from jax.experimental.pallas import tpu_sc as plsc

_LANES = 16
_JBLK = 8


@functools.lru_cache(maxsize=None)
def _build(batch, seq_len, vocab, hidden):
    info = plsc.get_sparse_core_info()
    num_workers = info.num_cores * info.num_subcores
    assert batch % num_workers == 0
    blk = batch // num_workers
    assert blk % 8 == 0
    ids_per_worker = blk * seq_len
    assert hidden % (_LANES * _JBLK) == 0
    vecs_per_row = hidden // _LANES

    mesh = plsc.VectorSubcoreMesh(core_axis_name="c", subcore_axis_name="s")

    nbuf = 4

    def body(
        ids_hbm, table_hbm, pos_hbm, out_hbm, idx_v, pos_v, buf, gsem, psem, ssem
    ):
        wid = lax.axis_index("s") * info.num_cores + lax.axis_index("c")
        b0 = wid * blk
        pltpu.sync_copy(
            ids_hbm.at[pl.ds(wid * ids_per_worker, ids_per_worker)], idx_v
        )

        def gather(l, slot):
            idx_slice = idx_v.at[pl.ds(l * blk, blk)]
            return pltpu.make_async_copy(
                table_hbm.at[idx_slice], buf.at[slot], gsem.at[slot]
            )

        def posrow(l, slot):
            return pltpu.make_async_copy(
                pos_hbm.at[l], pos_v.at[slot], psem.at[slot]
            )

        def scatter(l, slot):
            return pltpu.make_async_copy(
                buf.at[slot], out_hbm.at[l, pl.ds(b0, blk)], ssem.at[slot]
            )

        gather(0, 0).start()
        posrow(0, 0).start()
        gather(1, 1).start()
        posrow(1, 1).start()

        @pl.loop(0, seq_len)
        def pos_loop(l):
            slot = lax.rem(l, nbuf)
            slot2 = lax.rem(l + 2, nbuf)

            @pl.when(l >= 2)
            def _():
                scatter(l - 2, slot2).wait()

            @pl.when(l + 2 < seq_len)
            def _():
                gather(l + 2, slot2).start()
                posrow(l + 2, slot2).start()

            gather(l, slot).wait()
            posrow(l, slot).wait()

            for jb in range(vecs_per_row // _JBLK):
                pvs = [
                    pos_v[slot, pl.ds((jb * _JBLK + j) * _LANES, _LANES)]
                    for j in range(_JBLK)
                ]

                @plsc.parallel_loop(0, blk, unroll=2)
                def row_loop(i, pvs=pvs, jb=jb):
                    for j in range(_JBLK):
                        sl = pl.ds((jb * _JBLK + j) * _LANES, _LANES)
                        plsc.addupdate(buf.at[slot, i, sl], pvs[j])

            scatter(l, slot).start()

        scatter(seq_len - 2, lax.rem(seq_len - 2, nbuf)).wait()
        scatter(seq_len - 1, lax.rem(seq_len - 1, nbuf)).wait()

    return pl.kernel(
        body,
        out_type=jax.ShapeDtypeStruct((seq_len, batch, hidden), jnp.float32),
        mesh=mesh,
        scratch_types=[
            pltpu.VMEM((ids_per_worker,), jnp.int32),
            pltpu.VMEM((nbuf, hidden), jnp.float32),
            pltpu.VMEM((nbuf, blk, hidden), jnp.float32),
            pltpu.SemaphoreType.DMA((nbuf,)),
            pltpu.SemaphoreType.DMA((nbuf,)),
            pltpu.SemaphoreType.DMA((nbuf,)),
        ],
    )


def kernel(input_ids, token_table, pos_table):
    batch, seq_len = input_ids.shape
    vocab, hidden = token_table.shape
    assert seq_len == pos_table.shape[0]
    info = plsc.get_sparse_core_info()
    num_workers = info.num_cores * info.num_subcores
    blk = batch // num_workers
    ids = (
        input_ids.astype(jnp.int32)
        .reshape(num_workers, blk, seq_len)
        .transpose(0, 2, 1)
        .reshape(-1)
    )
    fn = _build(batch, seq_len, vocab, hidden)
    out = fn(ids, token_table, pos_table)
    return out.transpose(1, 0, 2)
